# Initial kernel scaffold; baseline (speedup 1.0000x reference)
import jax
import jax.numpy as jnp
from jax.experimental import pallas as pl


def kernel(x, inp_positions, out_positions):
    raise NotImplementedError("write your pallas kernel here")


# speedup vs baseline: 1.6391x; 1.6391x over previous
import functools

import jax
import jax.numpy as jnp
import numpy as np
from jax.experimental import pallas as pl


_R2 = np.float32(0.015 * 0.015)


def _ball_mean_block(o_ref, p_ref, xa_ref, out_ref):
    o = o_ref[...]
    p = p_ref[...]
    o2 = jnp.sum(o * o, axis=1)
    p2 = jnp.sum(p * p, axis=1)
    mm = jax.lax.dot_general(
        o, p, (((1,), (1,)), ((), ())),
        preferred_element_type=jnp.float32,
    )
    d2 = (o2[:, None] + p2[None, :]) - 2.0 * mm
    mask = (d2 <= _R2).astype(jnp.float32)
    res = jax.lax.dot_general(
        mask, xa_ref[...], (((1,), (0,)), ((), ())),
        preferred_element_type=jnp.float32,
    )
    num = res[:, :-1]
    dem = res[:, -1:]
    dem = jnp.where(dem > 0.0, dem, 1.0)
    out_ref[...] = num / dem


@jax.jit
def kernel(x, inp_positions, out_positions):
    b, n_in = x.shape
    n_out = out_positions.shape[0]
    block_m = 256
    xa = jnp.concatenate(
        [x.T, jnp.ones((n_in, 1), dtype=x.dtype)], axis=1)
    grid = (n_out // block_m,)
    out_t = pl.pallas_call(
        _ball_mean_block,
        grid=grid,
        in_specs=[
            pl.BlockSpec((block_m, 3), lambda i: (i, 0)),
            pl.BlockSpec((n_in, 3), lambda i: (0, 0)),
            pl.BlockSpec((n_in, b + 1), lambda i: (0, 0)),
        ],
        out_specs=pl.BlockSpec((block_m, b), lambda i: (i, 0)),
        out_shape=jax.ShapeDtypeStruct((n_out, b), x.dtype),
    )(out_positions, inp_positions, xa)
    return out_t.T
